# Initial kernel scaffold; baseline (speedup 1.0000x reference)
import jax
import jax.numpy as jnp
from jax.experimental import pallas as pl


def kernel(x, edge_index, edge_weight, W_msg, b_msg, W_upd, b_upd):
    raise NotImplementedError("write your pallas kernel here")


# speedup vs baseline: 3.4998x; 3.4998x over previous
import jax
import jax.numpy as jnp
import numpy as np
from jax import lax
from jax.experimental import pallas as pl
from jax.experimental.pallas import tpu as pltpu
from jax.experimental.pallas import tpu_sc as plsc

N = 10000
D = 128
E = 320000
NC = 2
NS = 16
NW = NC * NS
CH = 80
SB = 5
NSB = E // (NW * SB * CH)
NPAD = 10240
ROWS_PER_TILE = NPAD // NS
CROWS = NPAD // D
NB = 1024


def _lane_bcast(v16, i):
    idx = jnp.full((16, 1), i, dtype=jnp.int32)
    dnums = lax.GatherDimensionNumbers(
        offset_dims=(), collapsed_slice_dims=(0,), start_index_map=(0,))
    return lax.gather(v16, idx, dnums, slice_sizes=(1,),
                      mode=lax.GatherScatterMode.PROMISE_IN_BOUNDS)


def _precompute_body(x_ref, ws_ref, wd_ref, bm_ref, a_ref, b_ref):
    xv = x_ref[...]
    a_ref[...] = jnp.dot(xv, ws_ref[...], preferred_element_type=jnp.float32)
    b_ref[...] = (jnp.dot(xv, wd_ref[...], preferred_element_type=jnp.float32)
                  + bm_ref[...])


def _precompute(x, w_src, w_dst, b_msg2):
    return pl.pallas_call(
        _precompute_body,
        grid=(N // 1000,),
        in_specs=[
            pl.BlockSpec((1000, D), lambda i: (i, 0)),
            pl.BlockSpec((D, D), lambda i: (0, 0)),
            pl.BlockSpec((D, D), lambda i: (0, 0)),
            pl.BlockSpec((1, D), lambda i: (0, 0)),
        ],
        out_specs=[
            pl.BlockSpec((1000, D), lambda i: (i, 0)),
            pl.BlockSpec((1000, D), lambda i: (i, 0)),
        ],
        out_shape=[
            jax.ShapeDtypeStruct((N, D), jnp.float32),
            jax.ShapeDtypeStruct((N, D), jnp.float32),
        ],
    )(x, w_src, w_dst, b_msg2)


def _sc_body(a_hbm, b_hbm, w_hbm, eye_hbm, src_hbm, dst_hbm, ew_hbm, z_hbm,
             out_hbm, cnt_hbm,
             src_s, dst_s, col_s, crow_s, ew_s, w_v, a_v, b_v,
             acc, cacc, sem_a, sem_b):
    cid = lax.axis_index("c")
    sid = lax.axis_index("s")
    wid = sid * NC + cid
    r0 = sid * ROWS_PER_TILE

    pltpu.sync_copy(z_hbm, acc.at[pl.ds(r0, ROWS_PER_TILE)])

    @pl.when(sid == 0)
    def _():
        pltpu.sync_copy(z_hbm.at[pl.ds(0, CROWS)], cacc)

    pltpu.sync_copy(w_hbm, w_v)
    plsc.subcore_barrier()

    w_regs = [w_v[pl.ds(16 * j, 16)] for j in range(D // 16)]

    def sb_body(sb, carry):
        pltpu.sync_copy(src_hbm.at[wid, sb], src_s)
        pltpu.sync_copy(dst_hbm.at[wid, sb], dst_s)
        pltpu.sync_copy(ew_hbm.at[wid, sb], ew_s)

        def split_body(r, c2):
            for k in range(CH // 16):
                sl = pl.ds(16 * k, 16)
                dv = dst_s[r, sl]
                col_s[r, sl] = lax.bitwise_and(dv, jnp.int32(127))
                crow_s[r, sl] = lax.shift_right_logical(dv, 7)
            return c2
        lax.fori_loop(0, SB, split_body, 0)

        def chunk_body(cl, c2):
            ga = pltpu.async_copy(a_hbm.at[src_s.at[cl]], a_v, sem_a)
            gb = pltpu.async_copy(b_hbm.at[dst_s.at[cl]], b_v, sem_b)
            ga.wait()
            gb.wait()
            for g in range(CH // 16):
                ew16 = ew_s[cl, pl.ds(16 * g, 16)]
                for i in range(16):
                    e = 16 * g + i
                    ewb = _lane_bcast(ew16, i)
                    for j in range(D // 16):
                        sl = pl.ds(16 * j, 16)
                        a_v[e, sl] = jnp.maximum(
                            a_v[e, sl] + b_v[e, sl] + ewb * w_regs[j],
                            jnp.float32(0.0))
            gi = pltpu.async_copy(eye_hbm.at[col_s.at[cl]], b_v, sem_b)
            pltpu.sync_copy(a_v, acc.at[dst_s.at[cl]], add=True)
            gi.wait()
            pltpu.sync_copy(b_v, cacc.at[crow_s.at[cl]], add=True)
            return c2
        lax.fori_loop(0, SB, chunk_body, 0)
        return carry
    lax.fori_loop(0, NSB, sb_body, 0)

    plsc.subcore_barrier()
    pltpu.sync_copy(acc.at[pl.ds(r0, ROWS_PER_TILE)],
                    out_hbm.at[cid, pl.ds(r0, ROWS_PER_TILE)])

    @pl.when(sid == 0)
    def _():
        pltpu.sync_copy(cacc, cnt_hbm.at[cid])


def _sc_messages(a, b, w_row, eye, src4, dst4, ew4, zeros):
    mesh = plsc.VectorSubcoreMesh(core_axis_name="c", subcore_axis_name="s")
    f = pl.kernel(
        _sc_body,
        out_type=[
            jax.ShapeDtypeStruct((NC, NPAD, D), jnp.float32),
            jax.ShapeDtypeStruct((NC, CROWS, D), jnp.float32),
        ],
        mesh=mesh,
        scratch_types=[
            pltpu.VMEM((SB, CH), jnp.int32),
            pltpu.VMEM((SB, CH), jnp.int32),
            pltpu.VMEM((SB, CH), jnp.int32),
            pltpu.VMEM((SB, CH), jnp.int32),
            pltpu.VMEM((SB, CH), jnp.float32),
            pltpu.VMEM((D,), jnp.float32),
            pltpu.VMEM((CH, D), jnp.float32),
            pltpu.VMEM((CH, D), jnp.float32),
            pltpu.VMEM_SHARED((NPAD, D), jnp.float32),
            pltpu.VMEM_SHARED((CROWS, D), jnp.float32),
            pltpu.SemaphoreType.DMA,
            pltpu.SemaphoreType.DMA,
        ],
    )
    return f(a, b, w_row, eye, src4, dst4, ew4, zeros)


def _update_body(x_ref, p_ref, c_ref, wux_ref, wua_ref, bu_ref, o_ref):
    p = p_ref[...]
    s = p[0] + p[1]
    c = c_ref[...]
    cnt = jnp.maximum(c[0] + c[1], jnp.float32(1.0))
    agg = s / cnt
    h = (jnp.dot(x_ref[...], wux_ref[...], preferred_element_type=jnp.float32)
         + jnp.dot(agg, wua_ref[...], preferred_element_type=jnp.float32)
         + bu_ref[...])
    o_ref[...] = jnp.maximum(h, jnp.float32(0.0))


def _update(xp, partials, counts2, wu_x, wu_a, b_upd2):
    return pl.pallas_call(
        _update_body,
        grid=(NPAD // NB,),
        in_specs=[
            pl.BlockSpec((NB, D), lambda i: (i, 0)),
            pl.BlockSpec((NC, NB, D), lambda i: (0, i, 0)),
            pl.BlockSpec((NC, NB, 1), lambda i: (0, i, 0)),
            pl.BlockSpec((D, D), lambda i: (0, 0)),
            pl.BlockSpec((D, D), lambda i: (0, 0)),
            pl.BlockSpec((1, D), lambda i: (0, 0)),
        ],
        out_specs=pl.BlockSpec((NB, D), lambda i: (i, 0)),
        out_shape=jax.ShapeDtypeStruct((NPAD, D), jnp.float32),
    )(xp, partials, counts2, wu_x, wu_a, b_upd2)


def kernel(x, edge_index, edge_weight, W_msg, b_msg, W_upd, b_upd):
    src4 = edge_index[0].astype(jnp.int32).reshape(NW, NSB, SB, CH)
    dst4 = edge_index[1].astype(jnp.int32).reshape(NW, NSB, SB, CH)
    ew4 = edge_weight.reshape(NW, NSB, SB, CH)
    w_src = W_msg[:D]
    w_dst = W_msg[D:2 * D]
    w_row = W_msg[2 * D]
    eye = jnp.asarray(np.eye(D, dtype=np.float32))
    a, b = _precompute(x, w_src, w_dst, b_msg.reshape(1, D))
    zeros = jnp.zeros((ROWS_PER_TILE, D), dtype=jnp.float32)
    partials, counts = _sc_messages(a, b, w_row, eye, src4, dst4, ew4, zeros)
    counts2 = counts.reshape(NC, NPAD, 1)
    xp = jnp.pad(x, ((0, NPAD - N), (0, 0)))
    out = _update(xp, partials, counts2, W_upd[:D], W_upd[D:],
                  b_upd.reshape(1, D))
    return out[:N]


# speedup vs baseline: 4.1400x; 1.1829x over previous
import jax
import jax.numpy as jnp
from jax import lax
from jax.experimental import pallas as pl
from jax.experimental.pallas import tpu as pltpu
from jax.experimental.pallas import tpu_sc as plsc

N = 10000
D = 128
E = 320000
NC = 2
NS = 16
NW = NC * NS
CH = 64
CPT = 160
E2 = NW * CPT * CH
SB = 4
NSB = CPT // SB
NPAD = 10240
ROWS_PER_TILE = NPAD // NS
NB = 1024


def _lane_bcast(v16, i):
    idx = jnp.full((16, 1), i, dtype=jnp.int32)
    dnums = lax.GatherDimensionNumbers(
        offset_dims=(), collapsed_slice_dims=(0,), start_index_map=(0,))
    return lax.gather(v16, idx, dnums, slice_sizes=(1,),
                      mode=lax.GatherScatterMode.PROMISE_IN_BOUNDS)


def _precompute_body(x_ref, ws_ref, wd_ref, bm_ref, a_ref, b_ref):
    xv = x_ref[...]
    a_ref[...] = jnp.dot(xv, ws_ref[...], preferred_element_type=jnp.float32)
    b_ref[...] = (jnp.dot(xv, wd_ref[...], preferred_element_type=jnp.float32)
                  + bm_ref[...])


def _precompute(xp, w_src, w_dst, b_msg2):
    return pl.pallas_call(
        _precompute_body,
        grid=(NPAD // NB,),
        in_specs=[
            pl.BlockSpec((NB, D), lambda i: (i, 0)),
            pl.BlockSpec((D, D), lambda i: (0, 0)),
            pl.BlockSpec((D, D), lambda i: (0, 0)),
            pl.BlockSpec((1, D), lambda i: (0, 0)),
        ],
        out_specs=[
            pl.BlockSpec((NB, D), lambda i: (i, 0)),
            pl.BlockSpec((NB, D), lambda i: (i, 0)),
        ],
        out_shape=[
            jax.ShapeDtypeStruct((NPAD, D), jnp.float32),
            jax.ShapeDtypeStruct((NPAD, D), jnp.float32),
        ],
    )(xp, w_src, w_dst, b_msg2)


def _sc_body(a_hbm, b_hbm, w_hbm, src_hbm, dst_hbm, ew_hbm, z2_hbm, z1_hbm,
             out_hbm, cnt_hbm,
             isrc, idst, iew, w_v, a2, b2, cnt, acc,
             sa0, sb0, sa1, sb1, stg):
    cid = lax.axis_index("c")
    sid = lax.axis_index("s")
    wid = sid * NC + cid
    r0 = sid * ROWS_PER_TILE

    def issue_stage(s):
        p = lax.rem(s, 2)
        pltpu.async_copy(src_hbm.at[wid, s], isrc.at[p], stg)
        pltpu.async_copy(dst_hbm.at[wid, s], idst.at[p], stg)
        pltpu.async_copy(ew_hbm.at[wid, s], iew.at[p], stg)

    def wait_stage(s):
        p = lax.rem(s, 2)
        pltpu.make_async_copy(src_hbm.at[wid, s], isrc.at[p], stg).wait()
        pltpu.make_async_copy(dst_hbm.at[wid, s], idst.at[p], stg).wait()
        pltpu.make_async_copy(ew_hbm.at[wid, s], iew.at[p], stg).wait()

    def issue_gather(c):
        s = c // SB
        p = lax.rem(s, 2)
        l = lax.rem(c, SB)

        @pl.when(lax.rem(c, 2) == 0)
        def _():
            pltpu.async_copy(a_hbm.at[isrc.at[p, l]], a2.at[0], sa0)
            pltpu.async_copy(b_hbm.at[idst.at[p, l]], b2.at[0], sb0)

        @pl.when(lax.rem(c, 2) == 1)
        def _():
            pltpu.async_copy(a_hbm.at[isrc.at[p, l]], a2.at[1], sa1)
            pltpu.async_copy(b_hbm.at[idst.at[p, l]], b2.at[1], sb1)

    def wait_gather(c):
        s = c // SB
        p = lax.rem(s, 2)
        l = lax.rem(c, SB)

        @pl.when(lax.rem(c, 2) == 0)
        def _():
            pltpu.make_async_copy(a_hbm.at[isrc.at[p, l]], a2.at[0],
                                  sa0).wait()
            pltpu.make_async_copy(b_hbm.at[idst.at[p, l]], b2.at[0],
                                  sb0).wait()

        @pl.when(lax.rem(c, 2) == 1)
        def _():
            pltpu.make_async_copy(a_hbm.at[isrc.at[p, l]], a2.at[1],
                                  sa1).wait()
            pltpu.make_async_copy(b_hbm.at[idst.at[p, l]], b2.at[1],
                                  sb1).wait()

    pltpu.sync_copy(z2_hbm, acc.at[pl.ds(r0, ROWS_PER_TILE)])
    pltpu.sync_copy(z1_hbm, cnt)
    pltpu.sync_copy(w_hbm, w_v)
    w_regs = [w_v[pl.ds(16 * j, 16)] for j in range(D // 16)]
    ones16 = jnp.full((16,), 1.0, dtype=jnp.float32)

    pltpu.sync_copy(src_hbm.at[wid, 0], isrc.at[0])
    pltpu.sync_copy(dst_hbm.at[wid, 0], idst.at[0])
    pltpu.sync_copy(ew_hbm.at[wid, 0], iew.at[0])
    issue_stage(1)
    issue_gather(0)
    plsc.subcore_barrier()

    def do_chunk(c):
        s = c // SB
        p = lax.rem(s, 2)
        l = lax.rem(c, SB)
        par = lax.rem(c, 2)
        for g in range(CH // 16):
            sl16 = pl.ds(16 * g, 16)
            ew16 = iew[p, l, sl16]
            dv = idst[p, l, sl16]
            plsc.addupdate_scatter(cnt, [dv], ones16)
            for i in range(16):
                e = 16 * g + i
                ewb = _lane_bcast(ew16, i)
                for j in range(D // 16):
                    sl = pl.ds(16 * j, 16)
                    a2[par, e, sl] = jnp.maximum(
                        a2[par, e, sl] + b2[par, e, sl] + ewb * w_regs[j],
                        jnp.float32(0.0))

    def scatter_chunk(c):
        s = c // SB
        p = lax.rem(s, 2)
        l = lax.rem(c, SB)

        @pl.when(lax.rem(c, 2) == 0)
        def _():
            pltpu.sync_copy(a2.at[0], acc.at[idst.at[p, l]], add=True)

        @pl.when(lax.rem(c, 2) == 1)
        def _():
            pltpu.sync_copy(a2.at[1], acc.at[idst.at[p, l]], add=True)

    def ring_body(c, carry):
        nxt = c + 1
        s2 = nxt // SB

        @pl.when(nxt < CPT)
        def _():
            @pl.when(lax.rem(nxt, SB) == 0)
            def _():
                wait_stage(s2)
            issue_gather(nxt)

        wait_gather(c)
        do_chunk(c)
        scatter_chunk(c)

        @pl.when((lax.rem(nxt, SB) == 0) & (s2 + 1 < NSB) & (nxt < CPT))
        def _():
            issue_stage(s2 + 1)
        return carry

    lax.fori_loop(0, CPT, ring_body, 0)

    plsc.subcore_barrier()
    pltpu.sync_copy(acc.at[pl.ds(r0, ROWS_PER_TILE)],
                    out_hbm.at[cid, pl.ds(r0, ROWS_PER_TILE)])
    pltpu.sync_copy(cnt, cnt_hbm.at[cid, sid])


def _sc_messages(a, b, w_row, src4, dst4, ew4, zeros2, zeros1):
    mesh = plsc.VectorSubcoreMesh(core_axis_name="c", subcore_axis_name="s")
    f = pl.kernel(
        _sc_body,
        out_type=[
            jax.ShapeDtypeStruct((NC, NPAD, D), jnp.float32),
            jax.ShapeDtypeStruct((NC, NS, NPAD), jnp.float32),
        ],
        mesh=mesh,
        compiler_params=pltpu.CompilerParams(needs_layout_passes=False),
        scratch_types=[
            pltpu.VMEM((2, SB, CH), jnp.int32),
            pltpu.VMEM((2, SB, CH), jnp.int32),
            pltpu.VMEM((2, SB, CH), jnp.float32),
            pltpu.VMEM((D,), jnp.float32),
            pltpu.VMEM((2, CH, D), jnp.float32),
            pltpu.VMEM((2, CH, D), jnp.float32),
            pltpu.VMEM((NPAD,), jnp.float32),
            pltpu.VMEM_SHARED((NPAD, D), jnp.float32),
            pltpu.SemaphoreType.DMA,
            pltpu.SemaphoreType.DMA,
            pltpu.SemaphoreType.DMA,
            pltpu.SemaphoreType.DMA,
            pltpu.SemaphoreType.DMA,
        ],
    )
    return f(a, b, w_row, src4, dst4, ew4, zeros2, zeros1)


def _update_body(x_ref, p_ref, c_ref, wux_ref, wua_ref, bu_ref, o_ref):
    p = p_ref[...]
    s = p[0] + p[1]
    cnt = jnp.maximum(jnp.sum(c_ref[...], axis=0), jnp.float32(1.0))
    agg = s / cnt[:, None]
    h = (jnp.dot(x_ref[...], wux_ref[...], preferred_element_type=jnp.float32)
         + jnp.dot(agg, wua_ref[...], preferred_element_type=jnp.float32)
         + bu_ref[...])
    o_ref[...] = jnp.maximum(h, jnp.float32(0.0))


def _update(xp, partials, counts, wu_x, wu_a, b_upd2):
    return pl.pallas_call(
        _update_body,
        grid=(NPAD // NB,),
        in_specs=[
            pl.BlockSpec((NB, D), lambda i: (i, 0)),
            pl.BlockSpec((NC, NB, D), lambda i: (0, i, 0)),
            pl.BlockSpec((NW, NB), lambda i: (0, i)),
            pl.BlockSpec((D, D), lambda i: (0, 0)),
            pl.BlockSpec((D, D), lambda i: (0, 0)),
            pl.BlockSpec((1, D), lambda i: (0, 0)),
        ],
        out_specs=pl.BlockSpec((NB, D), lambda i: (i, 0)),
        out_shape=jax.ShapeDtypeStruct((NPAD, D), jnp.float32),
    )(xp, partials, counts, wu_x, wu_a, b_upd2)


def kernel(x, edge_index, edge_weight, W_msg, b_msg, W_upd, b_upd):
    pad = E2 - E
    fill = jnp.full((pad,), NPAD - 1, dtype=jnp.int32)
    src4 = jnp.concatenate([edge_index[0].astype(jnp.int32), fill]
                           ).reshape(NW, NSB, SB, CH)
    dst4 = jnp.concatenate([edge_index[1].astype(jnp.int32), fill]
                           ).reshape(NW, NSB, SB, CH)
    ew4 = jnp.concatenate(
        [edge_weight, jnp.zeros((pad,), dtype=jnp.float32)]
    ).reshape(NW, NSB, SB, CH)
    w_src = W_msg[:D]
    w_dst = W_msg[D:2 * D]
    w_row = W_msg[2 * D]
    xp = jnp.pad(x, ((0, NPAD - N), (0, 0)))
    a, b = _precompute(xp, w_src, w_dst, b_msg.reshape(1, D))
    zeros2 = jnp.zeros((ROWS_PER_TILE, D), dtype=jnp.float32)
    zeros1 = jnp.zeros((NPAD,), dtype=jnp.float32)
    partials, counts = _sc_messages(a, b, w_row, src4, dst4, ew4,
                                    zeros2, zeros1)
    out = _update(xp, partials, counts.reshape(NW, NPAD),
                  W_upd[:D], W_upd[D:], b_upd.reshape(1, D))
    return out[:N]


# speedup vs baseline: 4.2919x; 1.0367x over previous
import jax
import jax.numpy as jnp
from jax import lax
from jax.experimental import pallas as pl
from jax.experimental.pallas import tpu as pltpu
from jax.experimental.pallas import tpu_sc as plsc

N = 10000
D = 128
E = 320000
NC = 2
NS = 16
NW = NC * NS
CH = 64
CPT = 160
E2 = NW * CPT * CH
SB = 4
NSB = CPT // SB
NPAD = 10240
ROWS_PER_TILE = NPAD // NS
CROWS = NPAD // D
NB = 1024


def _lane_bcast(v16, i):
    idx = jnp.full((16, 1), i, dtype=jnp.int32)
    dnums = lax.GatherDimensionNumbers(
        offset_dims=(), collapsed_slice_dims=(0,), start_index_map=(0,))
    return lax.gather(v16, idx, dnums, slice_sizes=(1,),
                      mode=lax.GatherScatterMode.PROMISE_IN_BOUNDS)


def _precompute_body(x_ref, ws_ref, wd_ref, bm_ref, a_ref, b_ref):
    xv = x_ref[...]
    a_ref[...] = jnp.dot(xv, ws_ref[...], preferred_element_type=jnp.float32)
    b_ref[...] = (jnp.dot(xv, wd_ref[...], preferred_element_type=jnp.float32)
                  + bm_ref[...])


def _precompute(xp, w_src, w_dst, b_msg2):
    return pl.pallas_call(
        _precompute_body,
        grid=(NPAD // NB,),
        in_specs=[
            pl.BlockSpec((NB, D), lambda i: (i, 0)),
            pl.BlockSpec((D, D), lambda i: (0, 0)),
            pl.BlockSpec((D, D), lambda i: (0, 0)),
            pl.BlockSpec((1, D), lambda i: (0, 0)),
        ],
        out_specs=[
            pl.BlockSpec((NB, D), lambda i: (i, 0)),
            pl.BlockSpec((NB, D), lambda i: (i, 0)),
        ],
        out_shape=[
            jax.ShapeDtypeStruct((NPAD, D), jnp.float32),
            jax.ShapeDtypeStruct((NPAD, D), jnp.float32),
        ],
    )(xp, w_src, w_dst, b_msg2)


def _sc_body(a_hbm, b_hbm, w_hbm, eye_hbm, src_hbm, dst_hbm, ew_hbm, z2_hbm,
             out_hbm, cnt_hbm,
             isrc, idst, iew, col, crow, w_v, a2, b2, eyebuf, acc, cacc,
             sa0, sb0, sa1, sb1, stg, se):
    cid = lax.axis_index("c")
    sid = lax.axis_index("s")
    wid = sid * NC + cid
    r0 = sid * ROWS_PER_TILE

    def issue_stage(s):
        p = lax.rem(s, 2)
        pltpu.async_copy(src_hbm.at[wid, s], isrc.at[p], stg)
        pltpu.async_copy(dst_hbm.at[wid, s], idst.at[p], stg)
        pltpu.async_copy(ew_hbm.at[wid, s], iew.at[p], stg)

    def wait_stage(s):
        p = lax.rem(s, 2)
        pltpu.make_async_copy(src_hbm.at[wid, s], isrc.at[p], stg).wait()
        pltpu.make_async_copy(dst_hbm.at[wid, s], idst.at[p], stg).wait()
        pltpu.make_async_copy(ew_hbm.at[wid, s], iew.at[p], stg).wait()
        for r in range(SB):
            for k in range(CH // 16):
                sl = pl.ds(16 * k, 16)
                dv = idst[p, r, sl]
                col[p, r, sl] = lax.bitwise_and(dv, jnp.int32(127))
                crow[p, r, sl] = lax.shift_right_logical(dv, 7)

    def issue_eye(c):
        s = c // SB
        p = lax.rem(s, 2)
        l = lax.rem(c, SB)
        pltpu.async_copy(eye_hbm.at[col.at[p, l]], eyebuf, se)

    def wait_eye(c):
        s = c // SB
        p = lax.rem(s, 2)
        l = lax.rem(c, SB)
        pltpu.make_async_copy(eye_hbm.at[col.at[p, l]], eyebuf, se).wait()

    def issue_gather(c):
        s = c // SB
        p = lax.rem(s, 2)
        l = lax.rem(c, SB)

        @pl.when(lax.rem(c, 2) == 0)
        def _():
            pltpu.async_copy(a_hbm.at[isrc.at[p, l]], a2.at[0], sa0)
            pltpu.async_copy(b_hbm.at[idst.at[p, l]], b2.at[0], sb0)

        @pl.when(lax.rem(c, 2) == 1)
        def _():
            pltpu.async_copy(a_hbm.at[isrc.at[p, l]], a2.at[1], sa1)
            pltpu.async_copy(b_hbm.at[idst.at[p, l]], b2.at[1], sb1)

    def wait_gather(c):
        s = c // SB
        p = lax.rem(s, 2)
        l = lax.rem(c, SB)

        @pl.when(lax.rem(c, 2) == 0)
        def _():
            pltpu.make_async_copy(a_hbm.at[isrc.at[p, l]], a2.at[0],
                                  sa0).wait()
            pltpu.make_async_copy(b_hbm.at[idst.at[p, l]], b2.at[0],
                                  sb0).wait()

        @pl.when(lax.rem(c, 2) == 1)
        def _():
            pltpu.make_async_copy(a_hbm.at[isrc.at[p, l]], a2.at[1],
                                  sa1).wait()
            pltpu.make_async_copy(b_hbm.at[idst.at[p, l]], b2.at[1],
                                  sb1).wait()

    pltpu.sync_copy(z2_hbm, acc.at[pl.ds(r0, ROWS_PER_TILE)])

    @pl.when(sid == 0)
    def _():
        pltpu.sync_copy(z2_hbm.at[pl.ds(0, CROWS)], cacc)

    pltpu.sync_copy(w_hbm, w_v)
    w_regs = [w_v[pl.ds(16 * j, 16)] for j in range(D // 16)]

    pltpu.sync_copy(src_hbm.at[wid, 0], isrc.at[0])
    pltpu.sync_copy(dst_hbm.at[wid, 0], idst.at[0])
    pltpu.sync_copy(ew_hbm.at[wid, 0], iew.at[0])
    for r in range(SB):
        for k in range(CH // 16):
            sl = pl.ds(16 * k, 16)
            dv = idst[0, r, sl]
            col[0, r, sl] = lax.bitwise_and(dv, jnp.int32(127))
            crow[0, r, sl] = lax.shift_right_logical(dv, 7)
    issue_stage(1)
    issue_gather(0)
    issue_eye(0)
    plsc.subcore_barrier()

    def do_chunk(c):
        s = c // SB
        p = lax.rem(s, 2)
        l = lax.rem(c, SB)
        par = lax.rem(c, 2)
        for g in range(CH // 16):
            sl16 = pl.ds(16 * g, 16)
            ew16 = iew[p, l, sl16]
            for i in range(16):
                e = 16 * g + i
                ewb = _lane_bcast(ew16, i)
                for j in range(D // 16):
                    sl = pl.ds(16 * j, 16)
                    a2[par, e, sl] = jnp.maximum(
                        a2[par, e, sl] + b2[par, e, sl] + ewb * w_regs[j],
                        jnp.float32(0.0))

    def scatter_chunk(c):
        s = c // SB
        p = lax.rem(s, 2)
        l = lax.rem(c, SB)

        @pl.when(lax.rem(c, 2) == 0)
        def _():
            pltpu.sync_copy(a2.at[0], acc.at[idst.at[p, l]], add=True)

        @pl.when(lax.rem(c, 2) == 1)
        def _():
            pltpu.sync_copy(a2.at[1], acc.at[idst.at[p, l]], add=True)

        wait_eye(c)
        pltpu.sync_copy(eyebuf, cacc.at[crow.at[p, l]], add=True)

        @pl.when(c + 1 < CPT)
        def _():
            issue_eye(c + 1)

    def ring_body(c, carry):
        nxt = c + 1
        s2 = nxt // SB

        @pl.when(nxt < CPT)
        def _():
            @pl.when(lax.rem(nxt, SB) == 0)
            def _():
                wait_stage(s2)
            issue_gather(nxt)

        wait_gather(c)
        do_chunk(c)
        scatter_chunk(c)

        @pl.when((lax.rem(nxt, SB) == 0) & (s2 + 1 < NSB) & (nxt < CPT))
        def _():
            issue_stage(s2 + 1)
        return carry

    lax.fori_loop(0, CPT, ring_body, 0)

    plsc.subcore_barrier()
    pltpu.sync_copy(acc.at[pl.ds(r0, ROWS_PER_TILE)],
                    out_hbm.at[cid, pl.ds(r0, ROWS_PER_TILE)])

    @pl.when(sid == 0)
    def _():
        pltpu.sync_copy(cacc, cnt_hbm.at[cid])


def _sc_messages(a, b, w_row, eye, src4, dst4, ew4, zeros2):
    mesh = plsc.VectorSubcoreMesh(core_axis_name="c", subcore_axis_name="s")
    f = pl.kernel(
        _sc_body,
        out_type=[
            jax.ShapeDtypeStruct((NC, NPAD, D), jnp.float32),
            jax.ShapeDtypeStruct((NC, CROWS, D), jnp.float32),
        ],
        mesh=mesh,
        scratch_types=[
            pltpu.VMEM((2, SB, CH), jnp.int32),
            pltpu.VMEM((2, SB, CH), jnp.int32),
            pltpu.VMEM((2, SB, CH), jnp.float32),
            pltpu.VMEM((2, SB, CH), jnp.int32),
            pltpu.VMEM((2, SB, CH), jnp.int32),
            pltpu.VMEM((D,), jnp.float32),
            pltpu.VMEM((2, CH, D), jnp.float32),
            pltpu.VMEM((2, CH, D), jnp.float32),
            pltpu.VMEM((CH, D), jnp.float32),
            pltpu.VMEM_SHARED((NPAD, D), jnp.float32),
            pltpu.VMEM_SHARED((CROWS, D), jnp.float32),
            pltpu.SemaphoreType.DMA,
            pltpu.SemaphoreType.DMA,
            pltpu.SemaphoreType.DMA,
            pltpu.SemaphoreType.DMA,
            pltpu.SemaphoreType.DMA,
            pltpu.SemaphoreType.DMA,
        ],
    )
    return f(a, b, w_row, eye, src4, dst4, ew4, zeros2)


def _update_body(x_ref, p_ref, c_ref, wux_ref, wua_ref, bu_ref, o_ref):
    p = p_ref[...]
    s = p[0] + p[1]
    cnt = jnp.maximum(jnp.sum(c_ref[...], axis=0), jnp.float32(1.0))
    agg = s / cnt
    h = (jnp.dot(x_ref[...], wux_ref[...], preferred_element_type=jnp.float32)
         + jnp.dot(agg, wua_ref[...], preferred_element_type=jnp.float32)
         + bu_ref[...])
    o_ref[...] = jnp.maximum(h, jnp.float32(0.0))


def _update(xp, partials, counts, wu_x, wu_a, b_upd2):
    return pl.pallas_call(
        _update_body,
        grid=(NPAD // NB,),
        in_specs=[
            pl.BlockSpec((NB, D), lambda i: (i, 0)),
            pl.BlockSpec((NC, NB, D), lambda i: (0, i, 0)),
            pl.BlockSpec((NC, NB, 1), lambda i: (0, i, 0)),
            pl.BlockSpec((D, D), lambda i: (0, 0)),
            pl.BlockSpec((D, D), lambda i: (0, 0)),
            pl.BlockSpec((1, D), lambda i: (0, 0)),
        ],
        out_specs=pl.BlockSpec((NB, D), lambda i: (i, 0)),
        out_shape=jax.ShapeDtypeStruct((NPAD, D), jnp.float32),
    )(xp, partials, counts, wu_x, wu_a, b_upd2)


def kernel(x, edge_index, edge_weight, W_msg, b_msg, W_upd, b_upd):
    pad = E2 - E
    fill = jnp.full((pad,), NPAD - 1, dtype=jnp.int32)
    src4 = jnp.concatenate([edge_index[0].astype(jnp.int32), fill]
                           ).reshape(NW, NSB, SB, CH)
    dst4 = jnp.concatenate([edge_index[1].astype(jnp.int32), fill]
                           ).reshape(NW, NSB, SB, CH)
    ew4 = jnp.concatenate(
        [edge_weight, jnp.zeros((pad,), dtype=jnp.float32)]
    ).reshape(NW, NSB, SB, CH)
    w_src = W_msg[:D]
    w_dst = W_msg[D:2 * D]
    w_row = W_msg[2 * D]
    xp = jnp.pad(x, ((0, NPAD - N), (0, 0)))
    a, b = _precompute(xp, w_src, w_dst, b_msg.reshape(1, D))
    zeros2 = jnp.zeros((ROWS_PER_TILE, D), dtype=jnp.float32)
    eye = jnp.eye(D, dtype=jnp.float32)
    partials, counts = _sc_messages(a, b, w_row, eye, src4, dst4, ew4, zeros2)
    out = _update(xp, partials, counts.reshape(NC, NPAD, 1),
                  W_upd[:D], W_upd[D:], b_upd.reshape(1, D))
    return out[:N]
